# Initial kernel scaffold; baseline (speedup 1.0000x reference)
import jax
import jax.numpy as jnp
from jax.experimental import pallas as pl


def kernel(x, Wg, W1, W2):
    raise NotImplementedError("write your pallas kernel here")


# speedup vs baseline: 2.9272x; 2.9272x over previous
import jax
import jax.numpy as jnp
from jax.experimental import pallas as pl
from jax.experimental.pallas import tpu as pltpu

E = 8
K = 2
T = 2
D = 768
F = 768
N = 2048

BN = 1024


def _gates_for_task(logits):
    lane = jax.lax.broadcasted_iota(jnp.int32, logits.shape, 1)
    v1 = jnp.max(logits, axis=-1, keepdims=True)
    i1 = jnp.min(jnp.where(logits == v1, lane, E), axis=-1, keepdims=True)
    m1 = lane == i1
    l2 = jnp.where(m1, -jnp.inf, logits)
    v2 = jnp.max(l2, axis=-1, keepdims=True)
    i2 = jnp.min(jnp.where(l2 == v2, lane, E), axis=-1, keepdims=True)
    m2 = lane == i2
    e2 = jnp.exp(v2 - v1)
    denom = 1.0 + e2
    g1 = 1.0 / denom
    g2 = e2 / denom
    return jnp.where(m1, g1, 0.0) + jnp.where(m2, g2, 0.0)


def _body(x_ref, wg_ref, w1_ref, w2_ref, out_ref, gates_ref):
    e = pl.program_id(1)

    @pl.when(e == 0)
    def _init():
        xv = x_ref[...]
        for t in range(T):
            logits = jnp.dot(xv, wg_ref[t], preferred_element_type=jnp.float32)
            gates_ref[t] = _gates_for_task(logits)
        out_ref[...] = jnp.zeros_like(out_ref)

    xv = x_ref[...]
    h = jnp.dot(xv, w1_ref[0], preferred_element_type=jnp.float32)
    h = jnp.where(h >= 0, h, 0.01 * h)
    y = jnp.dot(h, w2_ref[0], preferred_element_type=jnp.float32)

    onehot = jax.lax.broadcasted_iota(jnp.int32, (BN, E), 1) == e
    for t in range(T):
        g = jnp.sum(jnp.where(onehot, gates_ref[t], 0.0), axis=-1, keepdims=True)
        out_ref[t] += g * y


def kernel(x, Wg, W1, W2):
    grid = (N // BN, E)
    return pl.pallas_call(
        _body,
        grid=grid,
        in_specs=[
            pl.BlockSpec((BN, D), lambda i, e: (i, 0)),
            pl.BlockSpec((T, D, E), lambda i, e: (0, 0, 0)),
            pl.BlockSpec((1, D, F), lambda i, e: (e, 0, 0)),
            pl.BlockSpec((1, F, D), lambda i, e: (e, 0, 0)),
        ],
        out_specs=pl.BlockSpec((T, BN, D), lambda i, e: (0, i, 0)),
        out_shape=jax.ShapeDtypeStruct((T, N, D), jnp.float32),
        scratch_shapes=[pltpu.VMEM((T, BN, E), jnp.float32)],
    )(x, Wg, W1, W2)
